# baseline (speedup 1.0000x reference)
import functools

import jax
import jax.numpy as jnp
from jax import lax
from jax.experimental import pallas as pl
from jax.experimental.pallas import tpu as pltpu
from jax.experimental.pallas import tpu_sc as plsc

NC = 2
NS = 16
L = 16
F = 4
G = NS // F
T = (NC * NS) // G
CH = 4000
CHN = 2000
UE = 10
U = 5

_MESH = dict(core_axis_name="c", subcore_axis_name="s", num_cores=NC,
             num_subcores=NS)
_PARAMS = dict(
    mesh=plsc.VectorSubcoreMesh(**_MESH),
    compiler_params=pltpu.CompilerParams(needs_layout_passes=False),
)


def _zero_refs(refs, n):
    z = jnp.zeros((L,), jnp.float32)

    @plsc.parallel_loop(0, n // L, 1, unroll=U)
    def _(i):
        for ref in refs:
            ref[pl.ds(i * L, L)] = z


def _ring(srcs_hbm, bufs2, sems2, base, npieces, body, ch):
    def start(p):
        b = p % 2
        off = base + p * ch
        for src, dst in zip(srcs_hbm, bufs2[b]):
            pltpu.async_copy(src.at[pl.ds(off, ch)], dst.at[pl.ds(0, ch)],
                             sems2[b])

    start(0)
    for p in range(npieces):
        b = p % 2
        if p + 1 < npieces:
            start(p + 1)
        off = base + p * ch
        for src, dst in zip(srcs_hbm, bufs2[b]):
            pltpu.make_async_copy(src.at[pl.ds(off, ch)],
                                  dst.at[pl.ds(0, ch)], sems2[b]).wait()
        body(p, *bufs2[b])


def _deg_body(col_hbm, ea_hbm, degp_hbm,
              cb0, eb0, cb1, eb1, dacc, sem0, sem1):
    E = col_hbm.shape[0]
    n = degp_hbm.shape[1]
    c = lax.axis_index("c")
    s = lax.axis_index("s")
    wid = c * NS + s
    per = E // (NC * NS)
    _zero_refs((dacc,), n)

    def piece(p, cb, eb):
        @plsc.parallel_loop(0, CHN // L, 1, unroll=U)
        def _(i):
            sl = pl.ds(i * L, L)
            plsc.addupdate_scatter(dacc, [cb[sl]], eb[sl])

    _ring((col_hbm, ea_hbm), ((cb0, eb0), (cb1, eb1)), (sem0, sem1),
          wid * per, per // CHN, piece, CHN)
    pltpu.sync_copy(dacc, degp_hbm.at[wid])


def _deg_partials(col, edge_attr, n):
    k = pl.kernel(
        _deg_body,
        out_type=jax.ShapeDtypeStruct((NC * NS, n), jnp.float32),
        scratch_types=[
            pltpu.VMEM((CHN,), jnp.int32),
            pltpu.VMEM((CHN,), jnp.float32),
            pltpu.VMEM((CHN,), jnp.int32),
            pltpu.VMEM((CHN,), jnp.float32),
            pltpu.VMEM((n,), jnp.float32),
            pltpu.SemaphoreType.DMA,
            pltpu.SemaphoreType.DMA,
        ],
        **_PARAMS,
    )
    return k(col, edge_attr)


def _edge_loop(row_hbm, col_hbm, norm_src, xrs, ags, bufs2, sems2, c, s, E):
    t_chunk = c * (T // NC) + s // G
    per = E // T

    def piece(p, rb, cb, nb):
        @plsc.parallel_loop(0, CH // L, 1, unroll=UE)
        def _(i):
            sl = pl.ds(i * L, L)
            r = rb[sl]
            cc = cb[sl]
            nv = nb[sl]
            for j in range(F):
                xv = plsc.load_gather(xrs[j], [r])
                plsc.addupdate_scatter(ags[j], [cc], nv * xv)

    _ring((row_hbm, col_hbm, norm_src), bufs2, sems2,
          t_chunk * per, per // CH, piece, CH)
    return t_chunk


def _layer1_body(row_hbm, col_hbm, ea_hbm, dinv_hbm, xT_hbm, norm_hbm, agg_hbm,
                 dinv_v, xr0, xr1, xr2, xr3, ag0, ag1, ag2, ag3,
                 rb0, cb0, nb0, rb1, cb1, nb1, wbuf, sem0, sem1, semx):
    E = row_hbm.shape[0]
    n = dinv_hbm.shape[0]
    c = lax.axis_index("c")
    s = lax.axis_index("s")
    xrs = (xr0, xr1, xr2, xr3)
    ags = (ag0, ag1, ag2, ag3)
    bufs2 = ((rb0, cb0, nb0), (rb1, cb1, nb1))
    sems2 = (sem0, sem1)
    g = s % G

    pltpu.async_copy(dinv_hbm, dinv_v, semx)
    for j in range(F):
        pltpu.async_copy(xT_hbm.at[g * F + j], xrs[j], semx)
    _zero_refs(ags, n)
    pltpu.make_async_copy(dinv_hbm, dinv_v, semx).wait()
    for j in range(F):
        pltpu.make_async_copy(xT_hbm.at[g * F + j], xrs[j], semx).wait()

    per_np = E // (2 * NS)
    base_np = c * (E // 2) + s * per_np

    def npiece(p, rb, cb, eb):
        @plsc.parallel_loop(0, CHN // L, 1, unroll=U)
        def _(i):
            sl = pl.ds(i * L, L)
            dr = plsc.load_gather(dinv_v, [rb[sl]])
            dc = plsc.load_gather(dinv_v, [cb[sl]])
            wbuf[sl] = dr * eb[sl] * dc
        pltpu.sync_copy(wbuf.at[pl.ds(0, CHN)],
                        norm_hbm.at[pl.ds(base_np + p * CHN, CHN)])

    _ring((row_hbm, col_hbm, ea_hbm), bufs2, sems2, base_np, per_np // CHN,
          npiece, CHN)
    plsc.subcore_barrier()

    t_chunk = _edge_loop(row_hbm, col_hbm, norm_hbm, xrs, ags, bufs2, sems2,
                         c, s, E)
    for j in range(F):
        pltpu.sync_copy(ags[j], agg_hbm.at[t_chunk, g * F + j])


def _layer2_body(row_hbm, col_hbm, norm_hbm, xT_hbm, agg_hbm,
                 xr0, xr1, xr2, xr3, ag0, ag1, ag2, ag3,
                 rb0, cb0, nb0, rb1, cb1, nb1, sem0, sem1, semx):
    E = row_hbm.shape[0]
    n = xT_hbm.shape[1]
    c = lax.axis_index("c")
    s = lax.axis_index("s")
    xrs = (xr0, xr1, xr2, xr3)
    ags = (ag0, ag1, ag2, ag3)
    bufs2 = ((rb0, cb0, nb0), (rb1, cb1, nb1))
    g = s % G
    for j in range(F):
        pltpu.async_copy(xT_hbm.at[g * F + j], xrs[j], semx)
    _zero_refs(ags, n)
    for j in range(F):
        pltpu.make_async_copy(xT_hbm.at[g * F + j], xrs[j], semx).wait()
    t_chunk = _edge_loop(row_hbm, col_hbm, norm_hbm, xrs, ags, bufs2,
                         (sem0, sem1), c, s, E)
    for j in range(F):
        pltpu.sync_copy(ags[j], agg_hbm.at[t_chunk, g * F + j])


def _sc_layer1(row, col, edge_attr, dinv, xT):
    E = row.shape[0]
    n = dinv.shape[0]
    vf = lambda shape: pltpu.VMEM(shape, jnp.float32)
    vi = lambda shape: pltpu.VMEM(shape, jnp.int32)
    k = pl.kernel(
        _layer1_body,
        out_type=(jax.ShapeDtypeStruct((E,), jnp.float32),
                  jax.ShapeDtypeStruct((T, NS, n), jnp.float32)),
        scratch_types=[vf((n,))] * 9 +
                      [vi((CH,)), vi((CH,)), vf((CH,)),
                       vi((CH,)), vi((CH,)), vf((CH,)), vf((CH,)),
                       pltpu.SemaphoreType.DMA, pltpu.SemaphoreType.DMA,
                       pltpu.SemaphoreType.DMA],
        **_PARAMS,
    )
    return k(row, col, edge_attr, dinv, xT)


def _sc_layer2(row, col, norm, xT):
    E = row.shape[0]
    n = xT.shape[1]
    vf = lambda shape: pltpu.VMEM(shape, jnp.float32)
    vi = lambda shape: pltpu.VMEM(shape, jnp.int32)
    k = pl.kernel(
        _layer2_body,
        out_type=jax.ShapeDtypeStruct((T, NS, n), jnp.float32),
        scratch_types=[vf((n,))] * 8 +
                      [vi((CH,)), vi((CH,)), vf((CH,)),
                       vi((CH,)), vi((CH,)), vf((CH,)),
                       pltpu.SemaphoreType.DMA, pltpu.SemaphoreType.DMA,
                       pltpu.SemaphoreType.DMA],
        **_PARAMS,
    )
    return k(row, col, norm, xT)


def _tc1_body(z_ref, w1_ref, degp_ref, xt_ref, dinv_ref, sn_ref):
    deg = jnp.sum(degp_ref[...], axis=0, keepdims=True) + 1.0
    dinv = lax.rsqrt(deg)
    dinv_ref[...] = dinv
    sn_ref[...] = dinv * dinv
    xt_ref[...] = lax.dot_general(
        w1_ref[...], z_ref[...], (((0,), (1,)), ((), ())),
        preferred_element_type=jnp.float32)


def _tc2_body(agg_ref, xt_ref, sn_ref, b1_ref, w2_ref, out_ref):
    pre = (jnp.sum(agg_ref[...], axis=0) + sn_ref[...] * xt_ref[...]
           + b1_ref[...])
    x1t = jnp.maximum(pre, 0.0)
    out_ref[...] = lax.dot_general(
        w2_ref[...], x1t, (((0,), (0,)), ((), ())),
        preferred_element_type=jnp.float32)


def _tc3_body(agg_ref, xt_ref, sn_ref, b2_ref, wc_ref, bc_ref, out_ref):
    x2t = (jnp.sum(agg_ref[...], axis=0) + sn_ref[...] * xt_ref[...]
           + b2_ref[...])
    logits = lax.dot_general(
        wc_ref[...], x2t, (((0,), (0,)), ((), ())),
        preferred_element_type=jnp.float32) + bc_ref[...]
    m = jnp.max(logits, axis=0, keepdims=True)
    y = logits - m
    lse = jnp.log(jnp.sum(jnp.exp(y), axis=0, keepdims=True))
    out_ref[...] = jnp.transpose(y - lse)


def kernel(z, edge_index, edge_attr, W1, b1, W2, b2, Wc, bc):
    n, d = z.shape
    h = W1.shape[1]
    ncls = Wc.shape[1]

    row = edge_index[0]
    col = edge_index[1]
    degp = _deg_partials(col, edge_attr, n)

    xt1T, dinv2d, selfnorm = pl.pallas_call(
        _tc1_body,
        out_shape=(jax.ShapeDtypeStruct((h, n), jnp.float32),
                   jax.ShapeDtypeStruct((1, n), jnp.float32),
                   jax.ShapeDtypeStruct((1, n), jnp.float32)),
    )(z, W1, degp)

    norm, agg1 = _sc_layer1(row, col, edge_attr,
                            jnp.reshape(dinv2d, (n,)), xt1T)

    xt2T = pl.pallas_call(
        _tc2_body,
        out_shape=jax.ShapeDtypeStruct((h, n), jnp.float32),
    )(agg1, xt1T, selfnorm, jnp.reshape(b1, (h, 1)), W2)

    agg2 = _sc_layer2(row, col, norm, xt2T)

    out = pl.pallas_call(
        _tc3_body,
        out_shape=jax.ShapeDtypeStruct((n, ncls), jnp.float32),
    )(agg2, xt2T, selfnorm, jnp.reshape(b2, (h, 1)), Wc,
      jnp.reshape(bc, (ncls, 1)))
    return out


# speedup vs baseline: 1.1849x; 1.1849x over previous
import jax
import jax.numpy as jnp
from jax import lax
from jax.experimental import pallas as pl
from jax.experimental.pallas import tpu as pltpu
from jax.experimental.pallas import tpu_sc as plsc

NC = 2
NS = 16
L = 16
F = 4
G = NS // F
T = (NC * NS) // G
CH = 4000
CHN = 2000
UE = 10
U = 5
BLK = 128
CHB = 13

_MESH = dict(core_axis_name="c", subcore_axis_name="s", num_cores=NC,
             num_subcores=NS)
_PARAMS = dict(
    mesh=plsc.VectorSubcoreMesh(**_MESH),
    compiler_params=pltpu.CompilerParams(needs_layout_passes=False),
)


def _zero_refs(refs, n):
    z = jnp.zeros((L,), jnp.float32)

    @plsc.parallel_loop(0, n // L, 1, unroll=U)
    def _(i):
        for ref in refs:
            ref[pl.ds(i * L, L)] = z


def _ring(srcs_hbm, bufs2, sems2, base, npieces, body, ch):
    def start(p):
        b = p % 2
        off = base + p * ch
        for src, dst in zip(srcs_hbm, bufs2[b]):
            pltpu.async_copy(src.at[pl.ds(off, ch)], dst.at[pl.ds(0, ch)],
                             sems2[b])

    start(0)
    for p in range(npieces):
        b = p % 2
        if p + 1 < npieces:
            start(p + 1)
        off = base + p * ch
        for src, dst in zip(srcs_hbm, bufs2[b]):
            pltpu.make_async_copy(src.at[pl.ds(off, ch)],
                                  dst.at[pl.ds(0, ch)], sems2[b]).wait()
        body(p, *bufs2[b])


def _deg_body(ei_hbm, ea_hbm, rc_hbm, degp_hbm,
              eb0, ab0, eb1, ab1, wb, dacc, sem0, sem1):
    E = ei_hbm.shape[1]
    n = degp_hbm.shape[1]
    c = lax.axis_index("c")
    s = lax.axis_index("s")
    wid = c * NS + s
    blocks = E // BLK
    bpt = blocks // (NC * NS)
    left = blocks - bpt * NC * NS
    base_blk = wid * bpt
    _zero_refs((dacc,), n)

    def chunk(eb, ab, off, ch):
        pltpu.sync_copy(ei_hbm.at[:, pl.ds(off, ch)], eb.at[:, pl.ds(0, ch)])
        pltpu.sync_copy(ea_hbm.at[pl.ds(off, ch)], ab.at[pl.ds(0, ch)])

        @plsc.parallel_loop(0, ch // L, 1, unroll=U)
        def _(i):
            sl = pl.ds(i * L, L)
            r = eb[0, sl]
            cc = eb[1, sl]
            wb[sl] = jnp.bitwise_or(r, lax.shift_left(cc, 16))
            plsc.addupdate_scatter(dacc, [cc], ab[sl])

        pltpu.sync_copy(wb.at[pl.ds(0, ch)], rc_hbm.at[pl.ds(off, ch)])

    npieces = bpt // CHB
    for p in range(npieces):
        b = (eb0, ab0) if p % 2 == 0 else (eb1, ab1)
        chunk(b[0], b[1], (base_blk + p * CHB) * BLK, CHB * BLK)

    @pl.when(wid < left)
    def _():
        chunk(eb0, ab0, (blocks - left + wid) * BLK, BLK)

    pltpu.sync_copy(dacc, degp_hbm.at[wid])


def _sc_deg(edge_index, edge_attr, n):
    E = edge_index.shape[1]
    k = pl.kernel(
        _deg_body,
        out_type=(jax.ShapeDtypeStruct((E,), jnp.int32),
                  jax.ShapeDtypeStruct((NC * NS, n), jnp.float32)),
        scratch_types=[
            pltpu.VMEM((2, CHB * BLK), jnp.int32),
            pltpu.VMEM((CHB * BLK,), jnp.float32),
            pltpu.VMEM((2, CHB * BLK), jnp.int32),
            pltpu.VMEM((CHB * BLK,), jnp.float32),
            pltpu.VMEM((CHB * BLK,), jnp.int32),
            pltpu.VMEM((n,), jnp.float32),
            pltpu.SemaphoreType.DMA,
            pltpu.SemaphoreType.DMA,
        ],
        **_PARAMS,
    )
    return k(edge_index, edge_attr)


def _edge_loop(rc_hbm, norm_src, xps, ags, bufs2, sems2, c, s, E):
    t_chunk = c * (T // NC) + s // G
    per = E // T

    def piece(p, rcb, nb):
        @plsc.parallel_loop(0, CH // L, 1, unroll=UE)
        def _(i):
            sl = pl.ds(i * L, L)
            rcv = rcb[sl]
            nv = nb[sl]
            r = jnp.bitwise_and(rcv, 0xFFFF)
            cc = lax.shift_right_logical(rcv, 16)
            for m in range(2):
                pv = plsc.load_gather(xps[m], [r])
                ab = plsc.bitcast(pv, jnp.bfloat16)
                lo, hi = plsc.unpack(ab, format=plsc.PackFormat.INTERLEAVED,
                                     preferred_element_type=jnp.float32)
                plsc.addupdate_scatter(ags[2 * m], [cc], nv * lo)
                plsc.addupdate_scatter(ags[2 * m + 1], [cc], nv * hi)

    _ring((rc_hbm, norm_src), bufs2, sems2, t_chunk * per, per // CH,
          piece, CH)
    return t_chunk


def _agg_writeback(ags, agg_hbm, t_chunk, g):
    for m in range(2):
        pltpu.sync_copy(ags[2 * m], agg_hbm.at[t_chunk, 2 * g + m])
        pltpu.sync_copy(ags[2 * m + 1], agg_hbm.at[t_chunk, 8 + 2 * g + m])


def _layer1_body(rc_hbm, ea_hbm, dinv_hbm, xp_hbm, norm_hbm, agg_hbm,
                 dinv_v, xp0, xp1, ag0, ag1, ag2, ag3,
                 rb0, nb0, rb1, nb1, wbuf, sem0, sem1, semx):
    E = rc_hbm.shape[0]
    n = dinv_hbm.shape[0]
    c = lax.axis_index("c")
    s = lax.axis_index("s")
    xps = (xp0, xp1)
    ags = (ag0, ag1, ag2, ag3)
    bufs2 = ((rb0, nb0), (rb1, nb1))
    sems2 = (sem0, sem1)
    g = s % G

    pltpu.async_copy(dinv_hbm, dinv_v, semx)
    for m in range(2):
        pltpu.async_copy(xp_hbm.at[2 * g + m], xps[m], semx)
    _zero_refs(ags, n)
    pltpu.make_async_copy(dinv_hbm, dinv_v, semx).wait()
    for m in range(2):
        pltpu.make_async_copy(xp_hbm.at[2 * g + m], xps[m], semx).wait()

    per_np = E // (2 * NS)
    base_np = c * (E // 2) + s * per_np

    def npiece(p, rb, eb):
        @plsc.parallel_loop(0, CHN // L, 1, unroll=U)
        def _(i):
            sl = pl.ds(i * L, L)
            rcv = rb[sl]
            r = jnp.bitwise_and(rcv, 0xFFFF)
            cc = lax.shift_right_logical(rcv, 16)
            dr = plsc.load_gather(dinv_v, [r])
            dc = plsc.load_gather(dinv_v, [cc])
            wbuf[sl] = dr * eb[sl] * dc
        pltpu.sync_copy(wbuf.at[pl.ds(0, CHN)],
                        norm_hbm.at[pl.ds(base_np + p * CHN, CHN)])

    _ring((rc_hbm, ea_hbm), bufs2, sems2, base_np, per_np // CHN,
          npiece, CHN)
    plsc.subcore_barrier()

    t_chunk = _edge_loop(rc_hbm, norm_hbm, xps, ags, bufs2, sems2, c, s, E)
    _agg_writeback(ags, agg_hbm, t_chunk, g)


def _layer2_body(rc_hbm, norm_hbm, xp_hbm, agg_hbm,
                 xp0, xp1, ag0, ag1, ag2, ag3,
                 rb0, nb0, rb1, nb1, sem0, sem1, semx):
    E = rc_hbm.shape[0]
    n = xp_hbm.shape[1]
    c = lax.axis_index("c")
    s = lax.axis_index("s")
    xps = (xp0, xp1)
    ags = (ag0, ag1, ag2, ag3)
    bufs2 = ((rb0, nb0), (rb1, nb1))
    g = s % G
    for m in range(2):
        pltpu.async_copy(xp_hbm.at[2 * g + m], xps[m], semx)
    _zero_refs(ags, n)
    for m in range(2):
        pltpu.make_async_copy(xp_hbm.at[2 * g + m], xps[m], semx).wait()
    t_chunk = _edge_loop(rc_hbm, norm_hbm, xps, ags, bufs2, (sem0, sem1),
                         c, s, E)
    _agg_writeback(ags, agg_hbm, t_chunk, g)


def _sc_layer1(rc, edge_attr, dinv, xp):
    E = rc.shape[0]
    n = dinv.shape[0]
    vf = lambda shape: pltpu.VMEM(shape, jnp.float32)
    vi = lambda shape: pltpu.VMEM(shape, jnp.int32)
    k = pl.kernel(
        _layer1_body,
        out_type=(jax.ShapeDtypeStruct((E,), jnp.float32),
                  jax.ShapeDtypeStruct((T, NS, n), jnp.float32)),
        scratch_types=[vf((n,)), vi((n,)), vi((n,))] + [vf((n,))] * 4 +
                      [vi((CH,)), vf((CH,)), vi((CH,)), vf((CH,)),
                       vf((CH,)),
                       pltpu.SemaphoreType.DMA, pltpu.SemaphoreType.DMA,
                       pltpu.SemaphoreType.DMA],
        **_PARAMS,
    )
    return k(rc, edge_attr, dinv, xp)


def _sc_layer2(rc, norm, xp):
    E = rc.shape[0]
    n = xp.shape[1]
    vf = lambda shape: pltpu.VMEM(shape, jnp.float32)
    vi = lambda shape: pltpu.VMEM(shape, jnp.int32)
    k = pl.kernel(
        _layer2_body,
        out_type=jax.ShapeDtypeStruct((T, NS, n), jnp.float32),
        scratch_types=[vi((n,)), vi((n,))] + [vf((n,))] * 4 +
                      [vi((CH,)), vf((CH,)), vi((CH,)), vf((CH,)),
                       pltpu.SemaphoreType.DMA, pltpu.SemaphoreType.DMA,
                       pltpu.SemaphoreType.DMA],
        **_PARAMS,
    )
    return k(rc, norm, xp)


def _pack_pairs(xe, xo):
    ie = lax.bitcast_convert_type(xe, jnp.int32) + 0x8000
    io = lax.bitcast_convert_type(xo, jnp.int32) + 0x8000
    lo = (ie.astype(jnp.uint32) >> 16).astype(jnp.int32)
    hi = jnp.bitwise_and(io, -65536)
    return jnp.bitwise_or(lo, hi)


def _dotT(w_ref, x):
    return lax.dot_general(w_ref[...], x, (((0,), (0,)), ((), ())),
                           preferred_element_type=jnp.float32)


def _tc1a_body(z_ref, w1e_ref, w1o_ref, xt_ref, xp_ref):
    zt = z_ref[...]
    xe = lax.dot_general(w1e_ref[...], zt, (((0,), (1,)), ((), ())),
                         preferred_element_type=jnp.float32)
    xo = lax.dot_general(w1o_ref[...], zt, (((0,), (1,)), ((), ())),
                         preferred_element_type=jnp.float32)
    xt_ref[...] = jnp.concatenate([xe, xo], axis=0)
    xp_ref[...] = _pack_pairs(xe, xo)


def _tc1b_body(degp_ref, dinv_ref, sn_ref):
    deg = jnp.sum(degp_ref[...], axis=0, keepdims=True) + 1.0
    dinv = lax.rsqrt(deg)
    dinv_ref[...] = dinv
    sn_ref[...] = dinv * dinv


def _tc2_body(agg_ref, xt_ref, sn_ref, b1_ref, w2e_ref, w2o_ref,
              xt2_ref, xp2_ref):
    pre = (jnp.sum(agg_ref[...], axis=0) + sn_ref[...] * xt_ref[...]
           + b1_ref[...])
    x1t = jnp.maximum(pre, 0.0)
    xe = _dotT(w2e_ref, x1t)
    xo = _dotT(w2o_ref, x1t)
    xt2_ref[...] = jnp.concatenate([xe, xo], axis=0)
    xp2_ref[...] = _pack_pairs(xe, xo)


def _tc3_body(agg_ref, xt_ref, sn_ref, b2_ref, wc_ref, bc_ref, out_ref):
    x2t = (jnp.sum(agg_ref[...], axis=0) + sn_ref[...] * xt_ref[...]
           + b2_ref[...])
    logits = _dotT(wc_ref, x2t) + bc_ref[...]
    m = jnp.max(logits, axis=0, keepdims=True)
    y = logits - m
    lse = jnp.log(jnp.sum(jnp.exp(y), axis=0, keepdims=True))
    out_ref[...] = jnp.transpose(y - lse)


def kernel(z, edge_index, edge_attr, W1, b1, W2, b2, Wc, bc):
    n, d = z.shape
    h = W1.shape[1]
    ncls = Wc.shape[1]
    hp = h // 2

    def permr(w):
        return jnp.concatenate([w[0::2], w[1::2]], axis=0)

    rc, degp = _sc_deg(edge_index, edge_attr, n)

    xt1T, xp1 = pl.pallas_call(
        _tc1a_body,
        out_shape=(jax.ShapeDtypeStruct((h, n), jnp.float32),
                   jax.ShapeDtypeStruct((hp, n), jnp.int32)),
    )(z, W1[:, 0::2], W1[:, 1::2])

    dinv2d, selfnorm = pl.pallas_call(
        _tc1b_body,
        out_shape=(jax.ShapeDtypeStruct((1, n), jnp.float32),
                   jax.ShapeDtypeStruct((1, n), jnp.float32)),
    )(degp)

    norm, agg1 = _sc_layer1(rc, edge_attr, jnp.reshape(dinv2d, (n,)), xp1)

    w2p = permr(W2)
    xt2T, xp2 = pl.pallas_call(
        _tc2_body,
        out_shape=(jax.ShapeDtypeStruct((h, n), jnp.float32),
                   jax.ShapeDtypeStruct((hp, n), jnp.int32)),
    )(agg1, xt1T, selfnorm, jnp.reshape(permr(b1), (h, 1)),
      w2p[:, 0::2], w2p[:, 1::2])

    agg2 = _sc_layer2(rc, norm, xp2)

    out = pl.pallas_call(
        _tc3_body,
        out_shape=jax.ShapeDtypeStruct((n, ncls), jnp.float32),
    )(agg2, xt2T, selfnorm, jnp.reshape(permr(b2), (h, 1)), permr(Wc),
      jnp.reshape(bc, (ncls, 1)))
    return out


# speedup vs baseline: 1.2499x; 1.0548x over previous
import jax
import jax.numpy as jnp
from jax import lax
from jax.experimental import pallas as pl
from jax.experimental.pallas import tpu as pltpu
from jax.experimental.pallas import tpu_sc as plsc

NC = 2
NS = 16
L = 16
F = 4
G = NS // F
T = (NC * NS) // G
CH = 4000
CHN = 2000
UE = 10
U = 5
BLK = 128
CHB = 13

_MESH = dict(core_axis_name="c", subcore_axis_name="s", num_cores=NC,
             num_subcores=NS)
_PARAMS = dict(
    mesh=plsc.VectorSubcoreMesh(**_MESH),
    compiler_params=pltpu.CompilerParams(needs_layout_passes=False),
)


def _zero_refs(refs, n):
    z = jnp.zeros((L,), jnp.float32)

    @plsc.parallel_loop(0, n // L, 1, unroll=U)
    def _(i):
        for ref in refs:
            ref[pl.ds(i * L, L)] = z


def _ring(srcs_hbm, bufs2, sems2, base, npieces, body, ch):
    def start(p):
        b = p % 2
        off = base + p * ch
        for src, dst in zip(srcs_hbm, bufs2[b]):
            pltpu.async_copy(src.at[pl.ds(off, ch)], dst.at[pl.ds(0, ch)],
                             sems2[b])

    start(0)
    for p in range(npieces):
        b = p % 2
        if p + 1 < npieces:
            start(p + 1)
        off = base + p * ch
        for src, dst in zip(srcs_hbm, bufs2[b]):
            pltpu.make_async_copy(src.at[pl.ds(off, ch)],
                                  dst.at[pl.ds(0, ch)], sems2[b]).wait()
        body(p, *bufs2[b])


def _deg_body(ei_hbm, ea_hbm, rc_hbm, degp_hbm,
              eb0, ab0, eb1, ab1, wb0, wb1, dacc,
              sem0, sem1, semw0, semw1):
    E = ei_hbm.shape[1]
    n = degp_hbm.shape[1]
    c = lax.axis_index("c")
    s = lax.axis_index("s")
    wid = c * NS + s
    blocks = E // BLK
    bpt = blocks // (NC * NS)
    left = blocks - bpt * NC * NS
    base_blk = wid * bpt
    _zero_refs((dacc,), n)

    CE = CHB * BLK
    ebs = (eb0, eb1)
    abs_ = (ab0, ab1)
    wbs = (wb0, wb1)
    sems = (sem0, sem1)
    semws = (semw0, semw1)
    npieces = bpt // CHB

    def off_of(p):
        return (base_blk + p * CHB) * BLK

    def start(p):
        b = p % 2
        pltpu.async_copy(ei_hbm.at[:, pl.ds(off_of(p), CE)], ebs[b], sems[b])
        pltpu.async_copy(ea_hbm.at[pl.ds(off_of(p), CE)], abs_[b], sems[b])

    start(0)
    for p in range(npieces):
        b = p % 2
        if p + 1 < npieces:
            start(p + 1)
        pltpu.make_async_copy(ei_hbm.at[:, pl.ds(off_of(p), CE)], ebs[b],
                              sems[b]).wait()
        pltpu.make_async_copy(ea_hbm.at[pl.ds(off_of(p), CE)], abs_[b],
                              sems[b]).wait()
        if p >= 2:
            pltpu.make_async_copy(wbs[b], rc_hbm.at[pl.ds(off_of(p - 2), CE)],
                                  semws[b]).wait()
        eb, ab, wb = ebs[b], abs_[b], wbs[b]

        @plsc.parallel_loop(0, CE // L, 1, unroll=U)
        def _(i):
            sl = pl.ds(i * L, L)
            r = eb[0, sl]
            cc = eb[1, sl]
            wb[sl] = jnp.bitwise_or(r, lax.shift_left(cc, 16))
            plsc.addupdate_scatter(dacc, [cc], ab[sl])

        pltpu.async_copy(wb, rc_hbm.at[pl.ds(off_of(p), CE)], semws[b])

    for p in (npieces - 2, npieces - 1):
        b = p % 2
        pltpu.make_async_copy(wbs[b], rc_hbm.at[pl.ds(off_of(p), CE)],
                              semws[b]).wait()

    @pl.when(wid < left)
    def _():
        off = (blocks - left + wid) * BLK
        pltpu.sync_copy(ei_hbm.at[:, pl.ds(off, BLK)],
                        eb0.at[:, pl.ds(0, BLK)])
        pltpu.sync_copy(ea_hbm.at[pl.ds(off, BLK)], ab0.at[pl.ds(0, BLK)])

        @plsc.parallel_loop(0, BLK // L, 1, unroll=U)
        def _(i):
            sl = pl.ds(i * L, L)
            r = eb0[0, sl]
            cc = eb0[1, sl]
            wb0[sl] = jnp.bitwise_or(r, lax.shift_left(cc, 16))
            plsc.addupdate_scatter(dacc, [cc], ab0[sl])

        pltpu.sync_copy(wb0.at[pl.ds(0, BLK)], rc_hbm.at[pl.ds(off, BLK)])

    pltpu.sync_copy(dacc, degp_hbm.at[wid])


def _sc_deg(edge_index, edge_attr, n):
    E = edge_index.shape[1]
    k = pl.kernel(
        _deg_body,
        out_type=(jax.ShapeDtypeStruct((E,), jnp.int32),
                  jax.ShapeDtypeStruct((NC * NS, n), jnp.float32)),
        scratch_types=[
            pltpu.VMEM((2, CHB * BLK), jnp.int32),
            pltpu.VMEM((CHB * BLK,), jnp.float32),
            pltpu.VMEM((2, CHB * BLK), jnp.int32),
            pltpu.VMEM((CHB * BLK,), jnp.float32),
            pltpu.VMEM((CHB * BLK,), jnp.int32),
            pltpu.VMEM((CHB * BLK,), jnp.int32),
            pltpu.VMEM((n,), jnp.float32),
            pltpu.SemaphoreType.DMA,
            pltpu.SemaphoreType.DMA,
            pltpu.SemaphoreType.DMA,
            pltpu.SemaphoreType.DMA,
        ],
        **_PARAMS,
    )
    return k(edge_index, edge_attr)


def _edge_loop(rc_hbm, norm_src, xps, ags, bufs2, sems2, c, s, E):
    t_chunk = c * (T // NC) + s // G
    per = E // T

    def piece(p, rcb, nb):
        @plsc.parallel_loop(0, CH // L, 1, unroll=UE)
        def _(i):
            sl = pl.ds(i * L, L)
            rcv = rcb[sl]
            nv = nb[sl]
            r = jnp.bitwise_and(rcv, 0xFFFF)
            cc = lax.shift_right_logical(rcv, 16)
            for m in range(2):
                pv = plsc.load_gather(xps[m], [r])
                ab = plsc.bitcast(pv, jnp.bfloat16)
                lo, hi = plsc.unpack(ab, format=plsc.PackFormat.INTERLEAVED,
                                     preferred_element_type=jnp.float32)
                plsc.addupdate_scatter(ags[2 * m], [cc], nv * lo)
                plsc.addupdate_scatter(ags[2 * m + 1], [cc], nv * hi)

    _ring((rc_hbm, norm_src), bufs2, sems2, t_chunk * per, per // CH,
          piece, CH)
    return t_chunk


def _agg_writeback(ags, agg_hbm, t_chunk, g):
    for m in range(2):
        pltpu.sync_copy(ags[2 * m], agg_hbm.at[t_chunk, 2 * g + m])
        pltpu.sync_copy(ags[2 * m + 1], agg_hbm.at[t_chunk, 8 + 2 * g + m])


def _layer1_body(rc_hbm, ea_hbm, dinv_hbm, xp_hbm, norm_hbm, agg_hbm,
                 dinv_v, xp0, xp1, ag0, ag1, ag2, ag3,
                 rb0, nb0, rb1, nb1, wbuf, sem0, sem1, semx):
    E = rc_hbm.shape[0]
    n = dinv_hbm.shape[0]
    c = lax.axis_index("c")
    s = lax.axis_index("s")
    xps = (xp0, xp1)
    ags = (ag0, ag1, ag2, ag3)
    bufs2 = ((rb0, nb0), (rb1, nb1))
    sems2 = (sem0, sem1)
    g = s % G

    pltpu.async_copy(dinv_hbm, dinv_v, semx)
    for m in range(2):
        pltpu.async_copy(xp_hbm.at[2 * g + m], xps[m], semx)
    _zero_refs(ags, n)
    pltpu.make_async_copy(dinv_hbm, dinv_v, semx).wait()
    for m in range(2):
        pltpu.make_async_copy(xp_hbm.at[2 * g + m], xps[m], semx).wait()

    per_np = E // (2 * NS)
    base_np = c * (E // 2) + s * per_np

    def npiece(p, rb, eb):
        @plsc.parallel_loop(0, CHN // L, 1, unroll=U)
        def _(i):
            sl = pl.ds(i * L, L)
            rcv = rb[sl]
            r = jnp.bitwise_and(rcv, 0xFFFF)
            cc = lax.shift_right_logical(rcv, 16)
            dr = plsc.load_gather(dinv_v, [r])
            dc = plsc.load_gather(dinv_v, [cc])
            wbuf[sl] = dr * eb[sl] * dc
        pltpu.sync_copy(wbuf.at[pl.ds(0, CHN)],
                        norm_hbm.at[pl.ds(base_np + p * CHN, CHN)])

    _ring((rc_hbm, ea_hbm), bufs2, sems2, base_np, per_np // CHN,
          npiece, CHN)
    plsc.subcore_barrier()

    t_chunk = _edge_loop(rc_hbm, norm_hbm, xps, ags, bufs2, sems2, c, s, E)
    _agg_writeback(ags, agg_hbm, t_chunk, g)


def _layer2_body(rc_hbm, norm_hbm, xp_hbm, agg_hbm,
                 xp0, xp1, ag0, ag1, ag2, ag3,
                 rb0, nb0, rb1, nb1, sem0, sem1, semx):
    E = rc_hbm.shape[0]
    n = xp_hbm.shape[1]
    c = lax.axis_index("c")
    s = lax.axis_index("s")
    xps = (xp0, xp1)
    ags = (ag0, ag1, ag2, ag3)
    bufs2 = ((rb0, nb0), (rb1, nb1))
    g = s % G
    for m in range(2):
        pltpu.async_copy(xp_hbm.at[2 * g + m], xps[m], semx)
    _zero_refs(ags, n)
    for m in range(2):
        pltpu.make_async_copy(xp_hbm.at[2 * g + m], xps[m], semx).wait()
    t_chunk = _edge_loop(rc_hbm, norm_hbm, xps, ags, bufs2, (sem0, sem1),
                         c, s, E)
    _agg_writeback(ags, agg_hbm, t_chunk, g)


def _sc_layer1(rc, edge_attr, dinv, xp):
    E = rc.shape[0]
    n = dinv.shape[0]
    vf = lambda shape: pltpu.VMEM(shape, jnp.float32)
    vi = lambda shape: pltpu.VMEM(shape, jnp.int32)
    k = pl.kernel(
        _layer1_body,
        out_type=(jax.ShapeDtypeStruct((E,), jnp.float32),
                  jax.ShapeDtypeStruct((T, NS, n), jnp.float32)),
        scratch_types=[vf((n,)), vi((n,)), vi((n,))] + [vf((n,))] * 4 +
                      [vi((CH,)), vf((CH,)), vi((CH,)), vf((CH,)),
                       vf((CH,)),
                       pltpu.SemaphoreType.DMA, pltpu.SemaphoreType.DMA,
                       pltpu.SemaphoreType.DMA],
        **_PARAMS,
    )
    return k(rc, edge_attr, dinv, xp)


def _sc_layer2(rc, norm, xp):
    E = rc.shape[0]
    n = xp.shape[1]
    vf = lambda shape: pltpu.VMEM(shape, jnp.float32)
    vi = lambda shape: pltpu.VMEM(shape, jnp.int32)
    k = pl.kernel(
        _layer2_body,
        out_type=jax.ShapeDtypeStruct((T, NS, n), jnp.float32),
        scratch_types=[vi((n,)), vi((n,))] + [vf((n,))] * 4 +
                      [vi((CH,)), vf((CH,)), vi((CH,)), vf((CH,)),
                       pltpu.SemaphoreType.DMA, pltpu.SemaphoreType.DMA,
                       pltpu.SemaphoreType.DMA],
        **_PARAMS,
    )
    return k(rc, norm, xp)


def _pack_pairs(xe, xo):
    ie = lax.bitcast_convert_type(xe, jnp.int32) + 0x8000
    io = lax.bitcast_convert_type(xo, jnp.int32) + 0x8000
    lo = (ie.astype(jnp.uint32) >> 16).astype(jnp.int32)
    hi = jnp.bitwise_and(io, -65536)
    return jnp.bitwise_or(lo, hi)


def _dotT(w_ref, x):
    return lax.dot_general(w_ref[...], x, (((0,), (0,)), ((), ())),
                           preferred_element_type=jnp.float32)


def _tc1a_body(z_ref, w1e_ref, w1o_ref, xt_ref, xp_ref):
    zt = z_ref[...]
    xe = lax.dot_general(w1e_ref[...], zt, (((0,), (1,)), ((), ())),
                         preferred_element_type=jnp.float32)
    xo = lax.dot_general(w1o_ref[...], zt, (((0,), (1,)), ((), ())),
                         preferred_element_type=jnp.float32)
    xt_ref[...] = jnp.concatenate([xe, xo], axis=0)
    xp_ref[...] = _pack_pairs(xe, xo)


def _tc1b_body(degp_ref, dinv_ref, sn_ref):
    deg = jnp.sum(degp_ref[...], axis=0, keepdims=True) + 1.0
    dinv = lax.rsqrt(deg)
    dinv_ref[...] = dinv
    sn_ref[...] = dinv * dinv


def _tc2_body(agg_ref, xt_ref, sn_ref, b1_ref, w2e_ref, w2o_ref,
              xt2_ref, xp2_ref):
    pre = (jnp.sum(agg_ref[...], axis=0) + sn_ref[...] * xt_ref[...]
           + b1_ref[...])
    x1t = jnp.maximum(pre, 0.0)
    xe = _dotT(w2e_ref, x1t)
    xo = _dotT(w2o_ref, x1t)
    xt2_ref[...] = jnp.concatenate([xe, xo], axis=0)
    xp2_ref[...] = _pack_pairs(xe, xo)


def _tc3_body(agg_ref, xt_ref, sn_ref, b2_ref, wc_ref, bc_ref, out_ref):
    x2t = (jnp.sum(agg_ref[...], axis=0) + sn_ref[...] * xt_ref[...]
           + b2_ref[...])
    logits = _dotT(wc_ref, x2t) + bc_ref[...]
    m = jnp.max(logits, axis=0, keepdims=True)
    y = logits - m
    lse = jnp.log(jnp.sum(jnp.exp(y), axis=0, keepdims=True))
    out_ref[...] = jnp.transpose(y - lse)


def kernel(z, edge_index, edge_attr, W1, b1, W2, b2, Wc, bc):
    n, d = z.shape
    h = W1.shape[1]
    ncls = Wc.shape[1]
    hp = h // 2

    def permr(w):
        return jnp.concatenate([w[0::2], w[1::2]], axis=0)

    rc, degp = _sc_deg(edge_index, edge_attr, n)

    xt1T, xp1 = pl.pallas_call(
        _tc1a_body,
        out_shape=(jax.ShapeDtypeStruct((h, n), jnp.float32),
                   jax.ShapeDtypeStruct((hp, n), jnp.int32)),
    )(z, W1[:, 0::2], W1[:, 1::2])

    dinv2d, selfnorm = pl.pallas_call(
        _tc1b_body,
        out_shape=(jax.ShapeDtypeStruct((1, n), jnp.float32),
                   jax.ShapeDtypeStruct((1, n), jnp.float32)),
    )(degp)

    norm, agg1 = _sc_layer1(rc, edge_attr, jnp.reshape(dinv2d, (n,)), xp1)

    w2p = permr(W2)
    xt2T, xp2 = pl.pallas_call(
        _tc2_body,
        out_shape=(jax.ShapeDtypeStruct((h, n), jnp.float32),
                   jax.ShapeDtypeStruct((hp, n), jnp.int32)),
    )(agg1, xt1T, selfnorm, jnp.reshape(permr(b1), (h, 1)),
      w2p[:, 0::2], w2p[:, 1::2])

    agg2 = _sc_layer2(rc, norm, xp2)

    out = pl.pallas_call(
        _tc3_body,
        out_shape=jax.ShapeDtypeStruct((n, ncls), jnp.float32),
    )(agg2, xt2T, selfnorm, jnp.reshape(permr(b2), (h, 1)), permr(Wc),
      jnp.reshape(bc, (ncls, 1)))
    return out


# speedup vs baseline: 1.3347x; 1.0679x over previous
import jax
import jax.numpy as jnp
from jax import lax
from jax.experimental import pallas as pl
from jax.experimental.pallas import tpu as pltpu
from jax.experimental.pallas import tpu_sc as plsc

NC = 2
NS = 16
L = 16
F = 4
G = NS // F
T = (NC * NS) // G
CH = 4000
CHN = 2000
UE = 10
U = 5
BLK = 128
CHB = 13

_MESH = dict(core_axis_name="c", subcore_axis_name="s", num_cores=NC,
             num_subcores=NS)
_PARAMS = dict(
    mesh=plsc.VectorSubcoreMesh(**_MESH),
    compiler_params=pltpu.CompilerParams(needs_layout_passes=False),
)


def _zero_refs(refs, n):
    z = jnp.zeros((L,), jnp.float32)

    @plsc.parallel_loop(0, n // L, 1, unroll=U)
    def _(i):
        for ref in refs:
            ref[pl.ds(i * L, L)] = z


def _ring(srcs_hbm, bufs2, sems2, base, npieces, body, ch):
    def start(p):
        b = p % 2
        off = base + p * ch
        for src, dst in zip(srcs_hbm, bufs2[b]):
            pltpu.async_copy(src.at[pl.ds(off, ch)], dst.at[pl.ds(0, ch)],
                             sems2[b])

    start(0)
    for p in range(npieces):
        b = p % 2
        if p + 1 < npieces:
            start(p + 1)
        off = base + p * ch
        for src, dst in zip(srcs_hbm, bufs2[b]):
            pltpu.make_async_copy(src.at[pl.ds(off, ch)],
                                  dst.at[pl.ds(0, ch)], sems2[b]).wait()
        body(p, *bufs2[b])


def _deg_body(ei_hbm, ea_hbm, rc_hbm, degp_hbm,
              eb0, ab0, eb1, ab1, wb0, wb1, dacc,
              sem0, sem1, semw0, semw1):
    E = ei_hbm.shape[1]
    n = degp_hbm.shape[1]
    c = lax.axis_index("c")
    s = lax.axis_index("s")
    wid = c * NS + s
    blocks = E // BLK
    bpt = blocks // (NC * NS)
    left = blocks - bpt * NC * NS
    base_blk = wid * bpt
    _zero_refs((dacc,), n)

    CE = CHB * BLK
    ebs = (eb0, eb1)
    abs_ = (ab0, ab1)
    wbs = (wb0, wb1)
    sems = (sem0, sem1)
    semws = (semw0, semw1)
    npieces = bpt // CHB

    def off_of(p):
        return (base_blk + p * CHB) * BLK

    def start(p):
        b = p % 2
        pltpu.async_copy(ei_hbm.at[:, pl.ds(off_of(p), CE)], ebs[b], sems[b])
        pltpu.async_copy(ea_hbm.at[pl.ds(off_of(p), CE)], abs_[b], sems[b])

    start(0)
    for p in range(npieces):
        b = p % 2
        if p + 1 < npieces:
            start(p + 1)
        pltpu.make_async_copy(ei_hbm.at[:, pl.ds(off_of(p), CE)], ebs[b],
                              sems[b]).wait()
        pltpu.make_async_copy(ea_hbm.at[pl.ds(off_of(p), CE)], abs_[b],
                              sems[b]).wait()
        if p >= 2:
            pltpu.make_async_copy(wbs[b], rc_hbm.at[pl.ds(off_of(p - 2), CE)],
                                  semws[b]).wait()
        eb, ab, wb = ebs[b], abs_[b], wbs[b]

        @plsc.parallel_loop(0, CE // L, 1, unroll=U)
        def _(i):
            sl = pl.ds(i * L, L)
            r = eb[0, sl]
            cc = eb[1, sl]
            wb[sl] = jnp.bitwise_or(r, lax.shift_left(cc, 16))
            plsc.addupdate_scatter(dacc, [cc], ab[sl])

        pltpu.async_copy(wb, rc_hbm.at[pl.ds(off_of(p), CE)], semws[b])

    for p in (npieces - 2, npieces - 1):
        b = p % 2
        pltpu.make_async_copy(wbs[b], rc_hbm.at[pl.ds(off_of(p), CE)],
                              semws[b]).wait()

    @pl.when(wid < left)
    def _():
        off = (blocks - left + wid) * BLK
        pltpu.sync_copy(ei_hbm.at[:, pl.ds(off, BLK)],
                        eb0.at[:, pl.ds(0, BLK)])
        pltpu.sync_copy(ea_hbm.at[pl.ds(off, BLK)], ab0.at[pl.ds(0, BLK)])

        @plsc.parallel_loop(0, BLK // L, 1, unroll=U)
        def _(i):
            sl = pl.ds(i * L, L)
            r = eb0[0, sl]
            cc = eb0[1, sl]
            wb0[sl] = jnp.bitwise_or(r, lax.shift_left(cc, 16))
            plsc.addupdate_scatter(dacc, [cc], ab0[sl])

        pltpu.sync_copy(wb0.at[pl.ds(0, BLK)], rc_hbm.at[pl.ds(off, BLK)])

    pltpu.sync_copy(dacc, degp_hbm.at[wid])


def _sc_deg(edge_index, edge_attr, n):
    E = edge_index.shape[1]
    k = pl.kernel(
        _deg_body,
        out_type=(jax.ShapeDtypeStruct((E,), jnp.int32),
                  jax.ShapeDtypeStruct((NC * NS, n), jnp.float32)),
        scratch_types=[
            pltpu.VMEM((2, CHB * BLK), jnp.int32),
            pltpu.VMEM((CHB * BLK,), jnp.float32),
            pltpu.VMEM((2, CHB * BLK), jnp.int32),
            pltpu.VMEM((CHB * BLK,), jnp.float32),
            pltpu.VMEM((CHB * BLK,), jnp.int32),
            pltpu.VMEM((CHB * BLK,), jnp.int32),
            pltpu.VMEM((n,), jnp.float32),
            pltpu.SemaphoreType.DMA,
            pltpu.SemaphoreType.DMA,
            pltpu.SemaphoreType.DMA,
            pltpu.SemaphoreType.DMA,
        ],
        **_PARAMS,
    )
    return k(edge_index, edge_attr)


def _edge_loop(rc_hbm, norm_src, xps, ags, bufs2, sems2, c, s, E):
    t_chunk = c * (T // NC) + s // G
    per = E // T

    def piece(p, rcb, nb):
        @plsc.parallel_loop(0, CH // L, 1, unroll=UE)
        def _(i):
            sl = pl.ds(i * L, L)
            rcv = rcb[sl]
            nv = nb[sl]
            r = jnp.bitwise_and(rcv, 0xFFFF)
            cc = lax.shift_right_logical(rcv, 16)
            for m in range(2):
                pv = plsc.load_gather(xps[m], [r])
                ab = plsc.bitcast(pv, jnp.bfloat16)
                lo, hi = plsc.unpack(ab, format=plsc.PackFormat.INTERLEAVED,
                                     preferred_element_type=jnp.float32)
                plsc.addupdate_scatter(ags[2 * m], [cc], nv * lo)
                plsc.addupdate_scatter(ags[2 * m + 1], [cc], nv * hi)

    _ring((rc_hbm, norm_src), bufs2, sems2, t_chunk * per, per // CH,
          piece, CH)
    return t_chunk


def _agg_writeback(ags, agg_hbm, t_chunk, g):
    for m in range(2):
        pltpu.sync_copy(ags[2 * m], agg_hbm.at[t_chunk, 2 * g + m])
        pltpu.sync_copy(ags[2 * m + 1], agg_hbm.at[t_chunk, 8 + 2 * g + m])


def _layer1_body(rc_hbm, ea_hbm, dinv_hbm, xp_hbm, norm_hbm, agg_hbm,
                 dinv_v, xp0, xp1, ag0, ag1, ag2, ag3,
                 rb0, nb0, rb1, nb1, wbuf, sem0, sem1, semx):
    E = rc_hbm.shape[0]
    n = dinv_hbm.shape[0]
    c = lax.axis_index("c")
    s = lax.axis_index("s")
    xps = (xp0, xp1)
    ags = (ag0, ag1, ag2, ag3)
    bufs2 = ((rb0, nb0), (rb1, nb1))
    sems2 = (sem0, sem1)
    g = s % G

    pltpu.async_copy(dinv_hbm, dinv_v, semx)
    for m in range(2):
        pltpu.async_copy(xp_hbm.at[2 * g + m], xps[m], semx)
    _zero_refs(ags, n)
    pltpu.make_async_copy(dinv_hbm, dinv_v, semx).wait()
    for m in range(2):
        pltpu.make_async_copy(xp_hbm.at[2 * g + m], xps[m], semx).wait()

    per_np = E // (2 * NS)
    base_np = c * (E // 2) + s * per_np

    def npiece(p, rb, eb):
        @plsc.parallel_loop(0, CHN // L, 1, unroll=U)
        def _(i):
            sl = pl.ds(i * L, L)
            rcv = rb[sl]
            r = jnp.bitwise_and(rcv, 0xFFFF)
            cc = lax.shift_right_logical(rcv, 16)
            dr = plsc.load_gather(dinv_v, [r])
            dc = plsc.load_gather(dinv_v, [cc])
            wbuf[sl] = dr * eb[sl] * dc
        pltpu.sync_copy(wbuf.at[pl.ds(0, CHN)],
                        norm_hbm.at[pl.ds(base_np + p * CHN, CHN)])

    _ring((rc_hbm, ea_hbm), bufs2, sems2, base_np, per_np // CHN,
          npiece, CHN)
    plsc.subcore_barrier()

    t_chunk = _edge_loop(rc_hbm, norm_hbm, xps, ags, bufs2, sems2, c, s, E)
    _agg_writeback(ags, agg_hbm, t_chunk, g)


def _layer2_body(rc_hbm, norm_hbm, xp_hbm, agg_hbm,
                 xp0, xp1, ag0, ag1, ag2, ag3,
                 rb0, nb0, rb1, nb1, sem0, sem1, semx):
    E = rc_hbm.shape[0]
    n = xp_hbm.shape[1]
    c = lax.axis_index("c")
    s = lax.axis_index("s")
    xps = (xp0, xp1)
    ags = (ag0, ag1, ag2, ag3)
    bufs2 = ((rb0, nb0), (rb1, nb1))
    g = s % G
    for m in range(2):
        pltpu.async_copy(xp_hbm.at[2 * g + m], xps[m], semx)
    _zero_refs(ags, n)
    for m in range(2):
        pltpu.make_async_copy(xp_hbm.at[2 * g + m], xps[m], semx).wait()
    t_chunk = _edge_loop(rc_hbm, norm_hbm, xps, ags, bufs2, (sem0, sem1),
                         c, s, E)
    _agg_writeback(ags, agg_hbm, t_chunk, g)


def _sc_layer1(rc, edge_attr, dinv, xp):
    E = rc.shape[0]
    n = dinv.shape[0]
    vf = lambda shape: pltpu.VMEM(shape, jnp.float32)
    vi = lambda shape: pltpu.VMEM(shape, jnp.int32)
    k = pl.kernel(
        _layer1_body,
        out_type=(jax.ShapeDtypeStruct((E,), jnp.float32),
                  jax.ShapeDtypeStruct((T, NS, n), jnp.float32)),
        scratch_types=[vf((n,)), vi((n,)), vi((n,))] + [vf((n,))] * 4 +
                      [vi((CH,)), vf((CH,)), vi((CH,)), vf((CH,)),
                       vf((CH,)),
                       pltpu.SemaphoreType.DMA, pltpu.SemaphoreType.DMA,
                       pltpu.SemaphoreType.DMA],
        **_PARAMS,
    )
    return k(rc, edge_attr, dinv, xp)


def _sc_layer2(rc, norm, xp):
    E = rc.shape[0]
    n = xp.shape[1]
    vf = lambda shape: pltpu.VMEM(shape, jnp.float32)
    vi = lambda shape: pltpu.VMEM(shape, jnp.int32)
    k = pl.kernel(
        _layer2_body,
        out_type=jax.ShapeDtypeStruct((T, NS, n), jnp.float32),
        scratch_types=[vi((n,)), vi((n,))] + [vf((n,))] * 4 +
                      [vi((CH,)), vf((CH,)), vi((CH,)), vf((CH,)),
                       pltpu.SemaphoreType.DMA, pltpu.SemaphoreType.DMA,
                       pltpu.SemaphoreType.DMA],
        **_PARAMS,
    )
    return k(rc, norm, xp)


def _pack_pairs(xe, xo):
    ie = lax.bitcast_convert_type(xe, jnp.int32) + 0x8000
    io = lax.bitcast_convert_type(xo, jnp.int32) + 0x8000
    lo = (ie.astype(jnp.uint32) >> 16).astype(jnp.int32)
    hi = jnp.bitwise_and(io, -65536)
    return jnp.bitwise_or(lo, hi)


def _dotT(w_ref, x):
    return lax.dot_general(w_ref[...], x, (((0,), (0,)), ((), ())),
                           preferred_element_type=jnp.float32)


def _tc1a_body(z_ref, w1e_ref, w1o_ref, xt_ref, xp_ref):
    zt = z_ref[...]
    xe = lax.dot_general(w1e_ref[...], zt, (((0,), (1,)), ((), ())),
                         preferred_element_type=jnp.float32)
    xo = lax.dot_general(w1o_ref[...], zt, (((0,), (1,)), ((), ())),
                         preferred_element_type=jnp.float32)
    xt_ref[...] = jnp.concatenate([xe, xo], axis=0)
    xp_ref[...] = _pack_pairs(xe, xo)


def _tc1b_body(degp_ref, dinv_ref, sn_ref):
    deg = jnp.sum(degp_ref[...], axis=0, keepdims=True) + 1.0
    dinv = lax.rsqrt(deg)
    dinv_ref[...] = dinv
    sn_ref[...] = dinv * dinv


def _tc2_body(agg_ref, xt_ref, sn_ref, b1_ref, w2e_ref, w2o_ref,
              xt2_ref, xp2_ref):
    pre = (jnp.sum(agg_ref[...], axis=0) + sn_ref[...] * xt_ref[...]
           + b1_ref[...])
    x1t = jnp.maximum(pre, 0.0)
    xe = _dotT(w2e_ref, x1t)
    xo = _dotT(w2o_ref, x1t)
    xt2_ref[...] = jnp.concatenate([xe, xo], axis=0)
    xp2_ref[...] = _pack_pairs(xe, xo)


def _tc3_body(agg_ref, xt_ref, sn_ref, b2_ref, wc_ref, bc_ref, out_ref):
    x2t = (jnp.sum(agg_ref[...], axis=0) + sn_ref[...] * xt_ref[...]
           + b2_ref[...])
    logits = _dotT(wc_ref, x2t) + bc_ref[...]
    m = jnp.max(logits, axis=0, keepdims=True)
    y = logits - m
    lse = jnp.log(jnp.sum(jnp.exp(y), axis=0, keepdims=True))
    out_ref[...] = y - lse


def kernel(z, edge_index, edge_attr, W1, b1, W2, b2, Wc, bc):
    n, d = z.shape
    h = W1.shape[1]
    ncls = Wc.shape[1]
    hp = h // 2

    def permr(w):
        return jnp.concatenate([w[0::2], w[1::2]], axis=0)

    rc, degp = _sc_deg(edge_index, edge_attr, n)

    xt1T, xp1 = pl.pallas_call(
        _tc1a_body,
        out_shape=(jax.ShapeDtypeStruct((h, n), jnp.float32),
                   jax.ShapeDtypeStruct((hp, n), jnp.int32)),
    )(z, W1[:, 0::2], W1[:, 1::2])

    dinv2d, selfnorm = pl.pallas_call(
        _tc1b_body,
        out_shape=(jax.ShapeDtypeStruct((1, n), jnp.float32),
                   jax.ShapeDtypeStruct((1, n), jnp.float32)),
    )(degp)

    norm, agg1 = _sc_layer1(rc, edge_attr, jnp.reshape(dinv2d, (n,)), xp1)

    w2p = permr(W2)
    xt2T, xp2 = pl.pallas_call(
        _tc2_body,
        out_shape=(jax.ShapeDtypeStruct((h, n), jnp.float32),
                   jax.ShapeDtypeStruct((hp, n), jnp.int32)),
    )(agg1, xt1T, selfnorm, jnp.reshape(permr(b1), (h, 1)),
      w2p[:, 0::2], w2p[:, 1::2])

    agg2 = _sc_layer2(rc, norm, xp2)

    outT = pl.pallas_call(
        _tc3_body,
        out_shape=jax.ShapeDtypeStruct((ncls, n), jnp.float32),
    )(agg2, xt2T, selfnorm, jnp.reshape(permr(b2), (h, 1)), permr(Wc),
      jnp.reshape(bc, (ncls, 1)))
    return jnp.transpose(outT)
